# baseline (speedup 1.0000x reference)
import jax
import jax.numpy as jnp
from jax.experimental import pallas as pl

_BLOCK_ROWS = 8


def _softmax_body(x_ref, o_ref):
    x = x_ref[...]
    m = jnp.max(x, axis=1, keepdims=True)
    e = jnp.exp(x - m)
    s = jnp.sum(e, axis=1, keepdims=True)
    o_ref[...] = e * (1.0 / s)


def kernel(logits):
    rows, cols = logits.shape
    br = _BLOCK_ROWS
    return pl.pallas_call(
        _softmax_body,
        grid=(rows // br,),
        in_specs=[pl.BlockSpec((br, cols), lambda i: (i, 0))],
        out_specs=pl.BlockSpec((br, cols), lambda i: (i, 0)),
        out_shape=jax.ShapeDtypeStruct((rows, cols), logits.dtype),
    )(logits)


# speedup vs baseline: 1.0598x; 1.0598x over previous
import jax
import jax.numpy as jnp
from jax.experimental import pallas as pl

_BLOCK_ROWS = 16


def _softmax_body(x_ref, o_ref):
    x = x_ref[...]
    m = jnp.max(x, axis=1, keepdims=True)
    e = jnp.exp(x - m)
    s = jnp.sum(e, axis=1, keepdims=True)
    o_ref[...] = e * (1.0 / s)


def kernel(logits):
    rows, cols = logits.shape
    br = _BLOCK_ROWS
    return pl.pallas_call(
        _softmax_body,
        grid=(rows // br,),
        in_specs=[pl.BlockSpec((br, cols), lambda i: (i, 0))],
        out_specs=pl.BlockSpec((br, cols), lambda i: (i, 0)),
        out_shape=jax.ShapeDtypeStruct((rows, cols), logits.dtype),
    )(logits)


# speedup vs baseline: 1.0619x; 1.0020x over previous
import jax
import jax.numpy as jnp
from jax.experimental import pallas as pl
from jax.experimental.pallas import tpu as pltpu

_BLOCK_ROWS = 8
_NBUF = 4


def _softmax_pipeline(x_hbm, o_hbm, in_bufs, out_bufs, in_sems, out_sems):
    rows, cols = x_hbm.shape
    br = _BLOCK_ROWS
    nblk = rows // br

    def in_copy(j, slot):
        return pltpu.make_async_copy(
            x_hbm.at[pl.ds(j * br, br), :], in_bufs.at[slot], in_sems.at[slot])

    def out_copy(j, slot):
        return pltpu.make_async_copy(
            out_bufs.at[slot], o_hbm.at[pl.ds(j * br, br), :], out_sems.at[slot])

    for j in range(min(_NBUF, nblk)):
        in_copy(j, j % _NBUF).start()

    for j in range(nblk):
        slot = j % _NBUF
        if j >= _NBUF:
            out_copy(j - _NBUF, slot).wait()
        in_copy(j, slot).wait()
        x = in_bufs[slot]
        m = jnp.max(x, axis=1, keepdims=True)
        e = jnp.exp(x - m)
        s = jnp.sum(e, axis=1, keepdims=True)
        out_bufs[slot] = e * (1.0 / s)
        out_copy(j, slot).start()
        if j + _NBUF < nblk:
            in_copy(j + _NBUF, slot).start()

    for j in range(max(nblk - _NBUF, 0), nblk):
        out_copy(j, j % _NBUF).wait()


def kernel(logits):
    rows, cols = logits.shape
    return pl.pallas_call(
        _softmax_pipeline,
        in_specs=[pl.BlockSpec(memory_space=pltpu.HBM)],
        out_specs=pl.BlockSpec(memory_space=pltpu.HBM),
        out_shape=jax.ShapeDtypeStruct((rows, cols), logits.dtype),
        scratch_shapes=[
            pltpu.VMEM((_NBUF, _BLOCK_ROWS, cols), jnp.float32),
            pltpu.VMEM((_NBUF, _BLOCK_ROWS, cols), jnp.float32),
            pltpu.SemaphoreType.DMA((_NBUF,)),
            pltpu.SemaphoreType.DMA((_NBUF,)),
        ],
    )(logits)
